# baseline (speedup 1.0000x reference)
import jax
import jax.numpy as jnp
from jax import lax
from jax.experimental import pallas as pl
from jax.experimental.pallas import tpu as pltpu

_BH = 32


def _make_body(d, h, w, c):
    nh = h // _BH
    steps = d * nh
    fr = w * c // 128

    def body(row_ref, dep_ref, colpat_ref, out_hbm, scratch, sems):
        s = pl.program_id(0)
        b = lax.rem(s, 2)
        di = lax.div(s, nh)
        hi = lax.rem(s, nh)
        out_flat = out_hbm

        def wait_for(ps):
            pb = lax.rem(ps, 2)
            pltpu.make_async_copy(
                scratch.at[pb], out_flat.at[ps], sems.at[pb]
            ).wait()

        @pl.when(s >= 2)
        def _():
            wait_for(s - 2)

        yblk = row_ref[pl.ds(hi * _BH, _BH), :]
        ytile = jnp.concatenate([yblk, yblk], axis=-1)
        zrow = dep_ref[pl.ds(di, 1), :]
        ztile = jnp.concatenate([zrow, zrow], axis=-1)

        r_i = lax.broadcasted_iota(jnp.int32, (fr, 128), 0)
        l_i = lax.broadcasted_iota(jnp.int32, (fr, 128), 1)
        m = 2 * r_i + l_i // 64
        t = lax.rem(m, 3)

        dense = jnp.broadcast_to(colpat_ref[...][None], (_BH, fr, 128))
        dense = jnp.where((t == 1)[None], ytile[:, None, :], dense)
        dense = jnp.where((t == 2)[None], ztile[:, None, :], dense)
        scratch[b] = dense

        pltpu.make_async_copy(
            scratch.at[b], out_flat.at[s], sems.at[b]
        ).start()

        @pl.when(s == steps - 1)
        def _():
            wait_for(s - 1)
            wait_for(s)

    return body


def kernel(scan, row_weight, col_weight, depth_weight):
    d, em, h, w = scan.shape
    c = row_weight.shape[1] + col_weight.shape[1] + depth_weight.shape[1]
    nh = h // _BH
    fr = w * c // 128
    cw = col_weight[:w]
    colpat = jnp.concatenate(
        [cw[:, None, :], jnp.zeros((w, 2, cw.shape[1]), cw.dtype)], axis=1
    ).reshape(fr, 128)
    flat = pl.pallas_call(
        _make_body(d, h, w, c),
        grid=(d * nh,),
        in_specs=[
            pl.BlockSpec(row_weight.shape, lambda s: (0, 0)),
            pl.BlockSpec(depth_weight.shape, lambda s: (0, 0)),
            pl.BlockSpec((fr, 128), lambda s: (0, 0)),
        ],
        out_specs=pl.BlockSpec(memory_space=pl.ANY),
        out_shape=jax.ShapeDtypeStruct((d * nh, _BH, fr, 128), jnp.float32),
        scratch_shapes=[
            pltpu.VMEM((2, _BH, fr, 128), jnp.float32),
            pltpu.SemaphoreType.DMA((2,)),
        ],
        compiler_params=pltpu.CompilerParams(
            dimension_semantics=("arbitrary",),
        ),
    )(row_weight, depth_weight, colpat)
    return jnp.reshape(flat, (d, h, w, c))


# speedup vs baseline: 3.8805x; 3.8805x over previous
import jax
import jax.numpy as jnp
from jax.experimental import pallas as pl
from jax.experimental.pallas import tpu as pltpu

_BH = 32
_K = 4
_RB = _BH // _K


def _make_body(d, h, w, c):
    nh = h // _BH
    steps = d * nh

    def body(row_ref, col_ref, dep_ref, out_hbm, scratch, sems):
        s = pl.program_id(0)
        b = jax.lax.rem(s, 2)
        di = jax.lax.div(s, nh)
        hi = jax.lax.rem(s, nh)

        def wait_for(ps):
            pb = jax.lax.rem(ps, 2)
            pdi = jax.lax.div(ps, nh)
            phi = jax.lax.rem(ps, nh)
            for k in range(_K):
                pltpu.make_async_copy(
                    scratch.at[pb, pl.ds(k * _RB, _RB)],
                    out_hbm.at[pdi, pl.ds(phi * _BH + k * _RB, _RB)],
                    sems.at[pb, k],
                ).wait()

        @pl.when(s >= 2)
        def _():
            wait_for(s - 2)

        x = col_ref[:w, :]
        y = row_ref[pl.ds(hi * _BH, _BH), :]
        z = dep_ref[pl.ds(di, 1), :]
        xb = jnp.broadcast_to(x[None, :, :], (_BH, w, 64))
        yb = jnp.broadcast_to(y[:, None, :], (_BH, w, 64))
        zb = jnp.broadcast_to(z[:, None, :], (_BH, w, 64))
        scratch[b] = jnp.concatenate([xb, yb, zb], axis=-1)

        for k in range(_K):
            pltpu.make_async_copy(
                scratch.at[b, pl.ds(k * _RB, _RB)],
                out_hbm.at[di, pl.ds(hi * _BH + k * _RB, _RB)],
                sems.at[b, k],
            ).start()

        @pl.when(s == steps - 1)
        def _():
            wait_for(s - 1)
            wait_for(s)

    return body


def kernel(scan, row_weight, col_weight, depth_weight):
    d, em, h, w = scan.shape
    c = row_weight.shape[1] + col_weight.shape[1] + depth_weight.shape[1]
    nh = h // _BH
    return pl.pallas_call(
        _make_body(d, h, w, c),
        grid=(d * nh,),
        in_specs=[
            pl.BlockSpec(row_weight.shape, lambda s: (0, 0)),
            pl.BlockSpec(col_weight.shape, lambda s: (0, 0)),
            pl.BlockSpec(depth_weight.shape, lambda s: (0, 0)),
        ],
        out_specs=pl.BlockSpec(memory_space=pl.ANY),
        out_shape=jax.ShapeDtypeStruct((d, h, w, c), jnp.float32),
        scratch_shapes=[
            pltpu.VMEM((2, _BH, w, c), jnp.float32),
            pltpu.SemaphoreType.DMA((2, _K)),
        ],
        compiler_params=pltpu.CompilerParams(
            dimension_semantics=("arbitrary",),
        ),
    )(row_weight, col_weight, depth_weight)
